# baseline (speedup 1.0000x reference)
import jax
import jax.numpy as jnp
from jax import lax
from jax.experimental import pallas as pl
from jax.experimental.pallas import tpu as pltpu

_C = 3
_NEG = 512
_T = 512
_MPAD = 256


def _body(aux_ref, kp_ref, bxt_ref, clsrow_ref, gcls_ref, greg_ref,
          cls_ref, reg_ref):
    kx = kp_ref[0, 0:1, :]
    ky = kp_ref[0, 1:2, :]
    kz = kp_ref[0, 2:3, :]
    neg = kp_ref[0, 3:4, :]

    cx = bxt_ref[0, :, 0:1]
    cy = bxt_ref[0, :, 1:2]
    cz = bxt_ref[0, :, 2:3]
    clsc = bxt_ref[0, :, 3:4]

    r0 = aux_ref[6, 0]
    r1 = aux_ref[6, 1]
    r2 = aux_ref[6, 2]
    rad = jnp.where(clsc == 0.0, r0, jnp.where(clsc == 1.0, r1, r2))

    dx = cx - kx
    dy = cy - ky
    dz = cz - kz
    dist = jnp.sqrt(dx * dx + dy * dy + dz * dz)
    ind = ((dist < rad) & (clsc >= 0.0)).astype(jnp.float32)

    clsrow = clsrow_ref[0, 0:1, :]
    riota = lax.broadcasted_iota(jnp.int32, (8, _MPAD), 0)
    w = ((riota == clsrow.astype(jnp.int32)) | (riota == 3)).astype(jnp.float32)
    cnt = lax.dot_general(w, ind, (((1,), (0,)), ((), ())),
                          preferred_element_type=jnp.float32)

    pf = [jnp.minimum(cnt[c:c + 1, :], 1.0) for c in range(_C)]
    nanyf = 1.0 - jnp.minimum(cnt[3:4, :], 1.0)
    colbg = neg * nanyf
    colig = (1.0 - neg) * nanyf
    zrow = jnp.zeros_like(kx)

    rows = pf + [pf[0] * kx, pf[0] * ky, pf[0] * kz,
                 pf[1] * kx, pf[1] * ky, pf[1] * kz,
                 pf[2] * kx, pf[2] * ky, pf[2] * kz,
                 colbg, colig, zrow, zrow]
    f = jnp.concatenate(rows, axis=0)

    dims = (((0,), (0,)), ((), ()))
    cls_ref[0] = lax.dot_general(f, gcls_ref[...], dims,
                                 preferred_element_type=jnp.float32)
    reg_ref[0] = lax.dot_general(f, greg_ref[...], dims,
                                 preferred_element_type=jnp.float32)


def kernel(keypoints, boxes, class_ids, anchor_sizes, anchor_radii):
    B, N, _ = keypoints.shape
    nb = boxes.shape[1]

    neg_inds = jax.random.randint(jax.random.key(1), (B, _NEG), 0, N)
    negmask = jnp.zeros((N,), jnp.float32).at[neg_inds.reshape(-1)].set(1.0)
    negmask = jnp.broadcast_to(negmask[None, None, :], (B, 1, N))

    kpt = jnp.concatenate([keypoints.transpose(0, 2, 1), negmask], axis=1)
    kpt = jnp.pad(kpt, ((0, 0), (0, 4), (0, 0)))

    clsf = class_ids.astype(jnp.float32)[..., None]
    bxt = jnp.concatenate([boxes[..., 0:3], clsf], axis=2)
    bxt = jnp.pad(bxt, ((0, 0), (0, _MPAD - nb), (0, 4)), constant_values=-1.0)

    clsrow = jnp.pad(class_ids.astype(jnp.float32)[:, None, :],
                     ((0, 0), (0, 7), (0, _MPAD - nb)), constant_values=-1.0)

    aux = jnp.zeros((8, 8), jnp.float32)
    aux = aux.at[6, 0:3].set(anchor_radii)

    fb = boxes.reshape(-1, 7)[0:_C]
    gcls = jnp.zeros((16, 5), jnp.float32)
    gcls = gcls.at[0, 0].set(1.0).at[1, 1].set(1.0).at[2, 2].set(1.0)
    gcls = gcls.at[12, 3].set(1.0).at[13, 4].set(1.0)
    greg = jnp.zeros((16, 28), jnp.float32)
    for c in range(_C):
        greg = greg.at[c, 7 * c + 0].set(fb[c, 0])
        greg = greg.at[c, 7 * c + 1].set(fb[c, 1])
        greg = greg.at[c, 7 * c + 2].set(fb[c, 2])
        greg = greg.at[3 + 3 * c + 0, 7 * c + 0].set(-1.0)
        greg = greg.at[3 + 3 * c + 1, 7 * c + 1].set(-1.0)
        greg = greg.at[3 + 3 * c + 2, 7 * c + 2].set(-1.0)
        ratio = (fb[c, 3:6] - anchor_sizes[c]) / anchor_sizes[c]
        greg = greg.at[c, 7 * c + 3:7 * c + 6].set(ratio)
        greg = greg.at[c, 7 * c + 6].set(fb[c, 6])

    clsf32, regf32 = pl.pallas_call(
        _body,
        grid=(B, pl.cdiv(N, _T)),
        in_specs=[
            pl.BlockSpec((8, 8), lambda b, n: (0, 0), memory_space=pltpu.SMEM),
            pl.BlockSpec((1, 8, _T), lambda b, n: (b, 0, n)),
            pl.BlockSpec((1, _MPAD, 8), lambda b, n: (b, 0, 0)),
            pl.BlockSpec((1, 8, _MPAD), lambda b, n: (b, 0, 0)),
            pl.BlockSpec((16, 5), lambda b, n: (0, 0)),
            pl.BlockSpec((16, 28), lambda b, n: (0, 0)),
        ],
        out_specs=[
            pl.BlockSpec((1, _T, 5), lambda b, n: (b, n, 0)),
            pl.BlockSpec((1, _T, 28), lambda b, n: (b, n, 0)),
        ],
        out_shape=[
            jax.ShapeDtypeStruct((B, N, 5), jnp.float32),
            jax.ShapeDtypeStruct((B, N, 28), jnp.float32),
        ],
    )(aux, kpt, bxt, clsrow, gcls, greg)

    targets_cls = clsf32.astype(bool)
    targets_reg = regf32.reshape(B, N, 4, 7)
    return targets_cls, targets_reg


# speedup vs baseline: 1.8972x; 1.8972x over previous
import functools

import numpy as np
import jax
import jax.numpy as jnp
from jax import lax
from jax.experimental import pallas as pl
from jax.experimental.pallas import tpu as pltpu

_C = 3
_NEG = 512
_T = 1024
_MPAD = 256


@functools.lru_cache(maxsize=None)
def _neg_mask(b, n):
    with jax.ensure_compile_time_eval():
        inds = jax.random.randint(jax.random.key(1), (b, _NEG), 0, n)
        mask = jnp.zeros((n,), jnp.float32).at[inds.reshape(-1)].set(1.0)
    return np.asarray(mask)


@functools.lru_cache(maxsize=None)
def _g_consts():
    gcls = np.zeros((16, 5), np.float32)
    gcls[0, 0] = gcls[1, 1] = gcls[2, 2] = gcls[12, 3] = gcls[13, 4] = 1.0
    glow = np.zeros((13, 28), np.float32)
    for c in range(_C):
        for a in range(3):
            glow[3 * c + a, 7 * c + a] = -1.0
    sel = np.zeros((_C, 4, 1), np.float32)
    for c in range(_C):
        sel[c, c, 0] = 1.0
    return gcls, glow, sel


def _body(rad_ref, kp_ref, bxt_ref, clsrow_ref, gcls_ref, greg_ref,
          cls_ref, reg_ref):
    kx = kp_ref[0, 0:1, :]
    ky = kp_ref[0, 1:2, :]
    kz = kp_ref[0, 2:3, :]
    neg = kp_ref[0, 3:4, :]

    cx = bxt_ref[0, :, 0:1]
    cy = bxt_ref[0, :, 1:2]
    cz = bxt_ref[0, :, 2:3]
    clsc = bxt_ref[0, :, 3:4]

    r0 = rad_ref[0, 0]
    r1 = rad_ref[0, 1]
    r2 = rad_ref[0, 2]
    rad = jnp.where(clsc == 0.0, r0,
                    jnp.where(clsc == 1.0, r1,
                              jnp.where(clsc == 2.0, r2, -1.0)))

    dx = cx - kx
    dy = cy - ky
    dz = cz - kz
    dist = jnp.sqrt(dx * dx + dy * dy + dz * dz)
    ind = jnp.where(dist < rad, 1.0, 0.0)

    clsrow = clsrow_ref[0, 0:1, :]
    riota = lax.broadcasted_iota(jnp.int32, (8, _MPAD), 0)
    w = ((riota == clsrow.astype(jnp.int32)) | (riota == 3)).astype(jnp.float32)
    cnt = lax.dot_general(w, ind, (((1,), (0,)), ((), ())),
                          preferred_element_type=jnp.float32)

    pf = [jnp.minimum(cnt[c:c + 1, :], 1.0) for c in range(_C)]
    nanyf = 1.0 - jnp.minimum(cnt[3:4, :], 1.0)
    colbg = neg * nanyf
    colig = (1.0 - neg) * nanyf
    zrow = jnp.zeros_like(kx)

    rows = pf + [pf[0] * kx, pf[0] * ky, pf[0] * kz,
                 pf[1] * kx, pf[1] * ky, pf[1] * kz,
                 pf[2] * kx, pf[2] * ky, pf[2] * kz,
                 colbg, colig, zrow, zrow]
    f = jnp.concatenate(rows, axis=0)

    dims = (((0,), (0,)), ((), ()))
    cls_ref[0] = lax.dot_general(f, gcls_ref[...], dims,
                                 preferred_element_type=jnp.float32)
    reg_ref[0] = lax.dot_general(f, greg_ref[...], dims,
                                 preferred_element_type=jnp.float32)


def kernel(keypoints, boxes, class_ids, anchor_sizes, anchor_radii):
    B, N, _ = keypoints.shape
    nb = boxes.shape[1]

    negmask = jnp.broadcast_to(jnp.asarray(_neg_mask(B, N))[None, None, :],
                               (B, 1, N))
    kpt = jnp.concatenate([keypoints.transpose(0, 2, 1), negmask], axis=1)
    kpt = jnp.pad(kpt, ((0, 0), (0, 4), (0, 0)))

    clsf = class_ids.astype(jnp.float32)[..., None]
    bxt = jnp.concatenate([boxes[..., 0:3], clsf], axis=2)
    bxt = jnp.pad(bxt, ((0, 0), (0, _MPAD - nb), (0, 4)), constant_values=-1.0)

    clsrow = jnp.pad(class_ids.astype(jnp.float32)[:, None, :],
                     ((0, 0), (0, 7), (0, _MPAD - nb)), constant_values=-1.0)

    rad = jnp.pad(anchor_radii, (0, 5))[None, :]

    gcls_np, glow_np, sel_np = _g_consts()
    fb = boxes.reshape(-1, 7)[0:_C]
    grow = jnp.concatenate(
        [fb[:, 0:3], (fb[:, 3:6] - anchor_sizes) / anchor_sizes, fb[:, 6:7]],
        axis=1)
    gtop = (grow[:, None, :] * jnp.asarray(sel_np)).reshape(_C, 28)
    greg = jnp.concatenate([gtop, jnp.asarray(glow_np)], axis=0)

    clsf32, regf32 = pl.pallas_call(
        _body,
        grid=(B, pl.cdiv(N, _T)),
        in_specs=[
            pl.BlockSpec((1, 8), lambda b, n: (0, 0), memory_space=pltpu.SMEM),
            pl.BlockSpec((1, 8, _T), lambda b, n: (b, 0, n)),
            pl.BlockSpec((1, _MPAD, 8), lambda b, n: (b, 0, 0)),
            pl.BlockSpec((1, 8, _MPAD), lambda b, n: (b, 0, 0)),
            pl.BlockSpec((16, 5), lambda b, n: (0, 0)),
            pl.BlockSpec((16, 28), lambda b, n: (0, 0)),
        ],
        out_specs=[
            pl.BlockSpec((1, _T, 5), lambda b, n: (b, n, 0)),
            pl.BlockSpec((1, _T, 28), lambda b, n: (b, n, 0)),
        ],
        out_shape=[
            jax.ShapeDtypeStruct((B, N, 5), jnp.float32),
            jax.ShapeDtypeStruct((B, N, 28), jnp.float32),
        ],
    )(rad, kpt, bxt, clsrow, jnp.asarray(gcls_np), greg)

    targets_cls = clsf32.astype(bool)
    targets_reg = regf32.reshape(B, N, 4, 7)
    return targets_cls, targets_reg
